# baseline (speedup 1.0000x reference)
import functools

import jax
import jax.numpy as jnp
from jax import lax
from jax.experimental import pallas as pl
from jax.experimental.pallas import tpu as pltpu
from jax.experimental.pallas import tpu_sc as plsc

B = 1024
N = 8192
F = 128
R_BLK = 512
K_BLK = 2048


def _agg_all_body(a_ref, c_ref, out_ref, acc_ref, *, nk):
    k = pl.program_id(1)

    @pl.when(k == 0)
    def _init():
        acc_ref[...] = jnp.zeros_like(acc_ref)

    m = a_ref[...].astype(jnp.bfloat16)
    acc_ref[...] += jnp.dot(m, c_ref[pl.ds(k * K_BLK, K_BLK), :],
                            preferred_element_type=jnp.float32)

    @pl.when(k == nk - 1)
    def _finish():
        acc = acc_ref[...]
        cnt = acc[:, F:F + 1]
        out_ref[...] = jnp.where(cnt > 0, acc[:, :F] / cnt, 0.0)


def _aggregate_all(adj, table):
    adj = adj.view(jnp.int8)
    table_bf = jnp.concatenate(
        [table.astype(jnp.bfloat16),
         jnp.ones((N, F), jnp.bfloat16)], axis=1)
    nk = N // K_BLK
    return pl.pallas_call(
        functools.partial(_agg_all_body, nk=nk),
        grid=(N // R_BLK, nk),
        in_specs=[
            pl.BlockSpec((R_BLK, K_BLK), lambda i, k: (i, k)),
            pl.BlockSpec((N, 2 * F), lambda i, k: (0, 0)),
        ],
        out_specs=pl.BlockSpec((R_BLK, F), lambda i, k: (i, 0)),
        out_shape=jax.ShapeDtypeStruct((N, F), jnp.float32),
        scratch_shapes=[pltpu.VMEM((R_BLK, 2 * F), jnp.float32)],
    )(adj, table_bf)


def _gather_rows_sc(p, idx):
    info = plsc.get_sparse_core_info()
    nc, ns = info.num_cores, info.num_subcores
    nw = nc * ns
    bpw = B // nw
    mesh = plsc.VectorSubcoreMesh(core_axis_name="c", subcore_axis_name="s")

    @functools.partial(
        pl.kernel,
        mesh=mesh,
        out_type=jax.ShapeDtypeStruct((B, F), jnp.float32),
        scratch_types=[
            pltpu.VMEM((bpw,), jnp.int32),
            pltpu.VMEM((bpw, F), jnp.float32),
            pltpu.SemaphoreType.DMA,
        ],
    )
    def gather(p_hbm, idx_hbm, out_hbm, idx_v, rows_v, sem):
        wid = lax.axis_index("s") * nc + lax.axis_index("c")
        base = wid * bpw
        pltpu.sync_copy(idx_hbm.at[pl.ds(base, bpw)], idx_v)
        pltpu.async_copy(p_hbm.at[idx_v], rows_v, sem).wait()
        pltpu.sync_copy(rows_v, out_hbm.at[pl.ds(base, bpw)])

    return gather(p, idx)


def _aggregate(adj, idx, table):
    return _gather_rows_sc(_aggregate_all(adj, table), idx)


def kernel(A_report_code, A_report_report, A_code_code, batch_nodes, R_table,
           C_table, gamma):
    idx = batch_nodes.astype(jnp.int32)
    code_emb = _aggregate(A_report_code, idx, C_table)

    def fast(code_emb):
        return code_emb

    def general(code_emb):
        report_emb = _aggregate(A_report_report, idx, R_table)
        return code_emb * gamma + report_emb * (1.0 - gamma)

    return jax.lax.cond(gamma[0] == 1.0, fast, general, code_emb)


# speedup vs baseline: 1.2349x; 1.2349x over previous
import functools

import jax
import jax.numpy as jnp
from jax import lax
from jax.experimental import pallas as pl
from jax.experimental.pallas import tpu as pltpu
from jax.experimental.pallas import tpu_sc as plsc

B = 1024
N = 8192
F = 128
CH = 512
NCH = N // CH
NBUF = 4


def _chunk_copy(a_hbm, scr, sems, chunk, buf):
    return pltpu.make_async_copy(
        a_hbm.at[pl.ds(chunk * CH, CH), :], scr.at[buf], sems.at[buf])


def _agg_all_body(c_ref, a_hbm, out_ref, scr, sems):
    i = pl.program_id(0)

    @pl.when(i == 0)
    def _prime():
        for b in range(NBUF):
            _chunk_copy(a_hbm, scr, sems, b, b).start()

    buf = i % NBUF
    _chunk_copy(a_hbm, scr, sems, i, buf).wait()
    m = scr[buf].astype(jnp.bfloat16)
    acc = jnp.dot(m, c_ref[...], preferred_element_type=jnp.float32)

    @pl.when(i < NCH - NBUF)
    def _next():
        _chunk_copy(a_hbm, scr, sems, i + NBUF, buf).start()

    cnt = acc[:, F:F + 1]
    out_ref[...] = jnp.where(cnt > 0, acc[:, :F] / cnt, 0.0)


def _aggregate_all(adj, table):
    adj = adj.view(jnp.int8)
    table_bf = jnp.concatenate(
        [table.astype(jnp.bfloat16),
         jnp.ones((N, F), jnp.bfloat16)], axis=1)
    return pl.pallas_call(
        _agg_all_body,
        grid=(NCH,),
        in_specs=[
            pl.BlockSpec((N, 2 * F), lambda i: (0, 0)),
            pl.BlockSpec(memory_space=pl.ANY),
        ],
        out_specs=pl.BlockSpec((CH, F), lambda i: (i, 0)),
        out_shape=jax.ShapeDtypeStruct((N, F), jnp.float32),
        scratch_shapes=[
            pltpu.VMEM((NBUF, CH, N), jnp.int8),
            pltpu.SemaphoreType.DMA((NBUF,)),
        ],
    )(table_bf, adj)


def _gather_rows_sc(p, idx):
    info = plsc.get_sparse_core_info()
    nc, ns = info.num_cores, info.num_subcores
    nw = nc * ns
    bpw = B // nw
    mesh = plsc.VectorSubcoreMesh(core_axis_name="c", subcore_axis_name="s")

    @functools.partial(
        pl.kernel,
        mesh=mesh,
        out_type=jax.ShapeDtypeStruct((B, F), jnp.float32),
        scratch_types=[
            pltpu.VMEM((bpw,), jnp.int32),
            pltpu.VMEM((bpw, F), jnp.float32),
            pltpu.SemaphoreType.DMA,
        ],
    )
    def gather(p_hbm, idx_hbm, out_hbm, idx_v, rows_v, sem):
        wid = lax.axis_index("s") * nc + lax.axis_index("c")
        base = wid * bpw
        pltpu.sync_copy(idx_hbm.at[pl.ds(base, bpw)], idx_v)
        pltpu.async_copy(p_hbm.at[idx_v], rows_v, sem).wait()
        pltpu.sync_copy(rows_v, out_hbm.at[pl.ds(base, bpw)])

    return gather(p, idx)


def _aggregate(adj, idx, table):
    return _gather_rows_sc(_aggregate_all(adj, table), idx)


def kernel(A_report_code, A_report_report, A_code_code, batch_nodes, R_table,
           C_table, gamma):
    idx = batch_nodes.astype(jnp.int32)
    code_emb = _aggregate(A_report_code, idx, C_table)

    def fast(code_emb):
        return code_emb

    def general(code_emb):
        report_emb = _aggregate(A_report_report, idx, R_table)
        return code_emb * gamma + report_emb * (1.0 - gamma)

    return jax.lax.cond(gamma[0] == 1.0, fast, general, code_emb)


# speedup vs baseline: 1.2355x; 1.0006x over previous
import functools

import jax
import jax.numpy as jnp
from jax import lax
from jax.experimental import pallas as pl
from jax.experimental.pallas import tpu as pltpu
from jax.experimental.pallas import tpu_sc as plsc

B = 1024
N = 8192
F = 128
CH = 512
NCH = N // CH
NBUF = 6


def _chunk_copy(a_hbm, scr, sems, chunk, buf):
    return pltpu.make_async_copy(
        a_hbm.at[pl.ds(chunk * CH, CH), :], scr.at[buf], sems.at[buf])


def _agg_all_body(chi_ref, a_hbm, out_ref, scr, sems):
    i = pl.program_id(0)

    @pl.when(i == 0)
    def _prime():
        for b in range(NBUF):
            _chunk_copy(a_hbm, scr, sems, b, b).start()

    buf = i % NBUF
    _chunk_copy(a_hbm, scr, sems, i, buf).wait()

    @pl.when(i < NCH - NBUF)
    def _next():
        _chunk_copy(a_hbm, scr, sems, i + NBUF, buf).start()

    m = scr[buf].astype(jnp.bfloat16)
    acc = jnp.dot(m, chi_ref[...], preferred_element_type=jnp.float32)
    cnt = acc[:, F:F + 1]
    out_ref[...] = jnp.where(cnt > 0, acc[:, :F] / cnt, 0.0)


def _aggregate_all(adj, table):
    adj = adj.view(jnp.int8)
    table_bf = jnp.concatenate(
        [table.astype(jnp.bfloat16),
         jnp.ones((N, F), jnp.bfloat16)], axis=1)
    return pl.pallas_call(
        _agg_all_body,
        grid=(NCH,),
        in_specs=[
            pl.BlockSpec((N, 2 * F), lambda i: (0, 0)),
            pl.BlockSpec(memory_space=pl.ANY),
        ],
        out_specs=pl.BlockSpec((CH, F), lambda i: (i, 0)),
        out_shape=jax.ShapeDtypeStruct((N, F), jnp.float32),
        scratch_shapes=[
            pltpu.VMEM((NBUF, CH, N), jnp.int8),
            pltpu.SemaphoreType.DMA((NBUF,)),
        ],
    )(table_bf, adj)


def _gather_rows_sc(p, idx):
    info = plsc.get_sparse_core_info()
    nc, ns = info.num_cores, info.num_subcores
    nw = nc * ns
    bpw = B // nw
    mesh = plsc.VectorSubcoreMesh(core_axis_name="c", subcore_axis_name="s")

    @functools.partial(
        pl.kernel,
        mesh=mesh,
        out_type=jax.ShapeDtypeStruct((B, F), jnp.float32),
        scratch_types=[
            pltpu.VMEM((bpw,), jnp.int32),
            pltpu.VMEM((bpw, F), jnp.float32),
            pltpu.SemaphoreType.DMA,
        ],
    )
    def gather(p_hbm, idx_hbm, out_hbm, idx_v, rows_v, sem):
        wid = lax.axis_index("s") * nc + lax.axis_index("c")
        base = wid * bpw
        pltpu.sync_copy(idx_hbm.at[pl.ds(base, bpw)], idx_v)
        pltpu.async_copy(p_hbm.at[idx_v], rows_v, sem).wait()
        pltpu.sync_copy(rows_v, out_hbm.at[pl.ds(base, bpw)])

    return gather(p, idx)


def _aggregate(adj, idx, table):
    return _gather_rows_sc(_aggregate_all(adj, table), idx)


def kernel(A_report_code, A_report_report, A_code_code, batch_nodes, R_table,
           C_table, gamma):
    idx = batch_nodes.astype(jnp.int32)
    code_emb = _aggregate(A_report_code, idx, C_table)

    def fast(code_emb):
        return code_emb

    def general(code_emb):
        report_emb = _aggregate(A_report_report, idx, R_table)
        return code_emb * gamma + report_emb * (1.0 - gamma)

    return jax.lax.cond(gamma[0] == 1.0, fast, general, code_emb)
